# baseline (speedup 1.0000x reference)
import functools

import jax
import jax.numpy as jnp
from jax import lax
from jax.experimental import pallas as pl
from jax.experimental.pallas import tpu as pltpu
from jax.experimental.pallas import tpu_sc as plsc

D = 128
SEQ = 200
TOK = 1024 * SEQ
NC, NS = 2, 16
NW = NC * NS
TPW = TOK // NW
C = 64
NCHUNK = TPW // C
LN_EPS = 1e-12
KVEC = D // 16


def _ln_body(ids, tts, word, pos, typ, gamma, beta, out,
             comb, typev, rows0, rows1, obuf0, obuf1, idxall, ttall, gv, bv,
             gsem0, gsem1, osem0, osem1):
    wid = lax.axis_index("s") * NC + lax.axis_index("c")
    base0 = wid * TPW

    pltpu.sync_copy(ids.at[pl.ds(base0, TPW)], idxall)
    pltpu.sync_copy(tts.at[pl.ds(base0, TPW)], ttall)
    pltpu.sync_copy(pos.at[pl.ds(0, SEQ)], comb.at[pl.ds(0, SEQ)])
    pltpu.sync_copy(pos.at[pl.ds(0, SEQ)], comb.at[pl.ds(SEQ, SEQ)])
    pltpu.sync_copy(typ, typev)
    pltpu.sync_copy(gamma, gv)
    pltpu.sync_copy(beta, bv)

    @plsc.parallel_loop(0, SEQ, 1, unroll=2)
    def add_type(l):
        for k in range(KVEC):
            s = pl.ds(k * 16, 16)
            comb[l, s] = comb[l, s] + typev[0, s]
            comb[SEQ + l, s] = comb[SEQ + l, s] + typev[1, s]

    lanes = lax.iota(jnp.int32, 16)
    rowss, obufs = (rows0, rows1), (obuf0, obuf1)
    gsems, osems = (gsem0, gsem1), (osem0, osem1)

    def compute_chunk(rbuf, obuf, coff):
        pos0 = lax.rem(base0 + coff, SEQ)

        @plsc.parallel_loop(0, C, 1, unroll=4)
        def tok_body(t):
            tsplat = jnp.full((16,), coff + t, jnp.int32)
            ttk = plsc.load_gather(ttall, [tsplat])
            p = lax.rem(pos0 + t, SEQ)
            arow = ttk * SEQ + p
            xs = []
            for k in range(KVEC):
                ad = plsc.load_gather(comb, [arow, lanes + (k * 16)])
                xs.append(rbuf[t, pl.ds(k * 16, 16)] + ad)
            s = xs[0]
            for k in range(1, KVEC):
                s = s + xs[k]
            sq = xs[0] * xs[0]
            for k in range(1, KVEC):
                sq = sq + xs[k] * xs[k]
            totv = jnp.full((16,), jnp.sum(s))
            tot2v = jnp.full((16,), jnp.sum(sq))
            meanv = totv * (1.0 / D)
            varv = tot2v * (1.0 / D) - meanv * meanv + LN_EPS
            yi = jnp.int32(0x5F3759DF) - lax.shift_right_arithmetic(
                plsc.bitcast(varv, jnp.int32), 1)
            y = plsc.bitcast(yi, jnp.float32)
            for _ in range(3):
                y = y * (1.5 - 0.5 * varv * y * y)
            for k in range(KVEC):
                s16 = pl.ds(k * 16, 16)
                obuf[t, s16] = (xs[k] - meanv) * (gv[s16] * y) + bv[s16]

    for b in range(2):
        pltpu.async_copy(word.at[idxall.at[pl.ds(b * C, C)]], rowss[b], gsems[b])

    def outer(cc, carry):
        for b in range(2):
            coff = (cc * 2 + b) * C
            pltpu.make_async_copy(
                word.at[idxall.at[pl.ds(coff, C)]], rowss[b], gsems[b]).wait()

            @pl.when(cc >= 1)
            def _():
                pltpu.make_async_copy(
                    obufs[b], out.at[pl.ds(base0 + coff, C)], osems[b]).wait()

            compute_chunk(rowss[b], obufs[b], coff)
            pltpu.async_copy(obufs[b], out.at[pl.ds(base0 + coff, C)], osems[b])

            @pl.when(cc <= NCHUNK // 2 - 2)
            def _():
                pltpu.async_copy(
                    word.at[idxall.at[pl.ds(coff + 2 * C, C)]],
                    rowss[b], gsems[b])
        return carry

    lax.fori_loop(0, NCHUNK // 2, outer, 0)

    for b in range(2):
        coff = (NCHUNK - 2 + b) * C
        pltpu.make_async_copy(
            obufs[b], out.at[pl.ds(base0 + coff, C)], osems[b]).wait()


_ln_kernel = functools.partial(
    pl.kernel,
    out_type=jax.ShapeDtypeStruct((TOK, D), jnp.float32),
    mesh=plsc.VectorSubcoreMesh(core_axis_name="c", subcore_axis_name="s"),
    scratch_types=[
        pltpu.VMEM((2 * SEQ, D), jnp.float32),
        pltpu.VMEM((2, D), jnp.float32),
        pltpu.VMEM((C, D), jnp.float32),
        pltpu.VMEM((C, D), jnp.float32),
        pltpu.VMEM((C, D), jnp.float32),
        pltpu.VMEM((C, D), jnp.float32),
        pltpu.VMEM((TPW,), jnp.int32),
        pltpu.VMEM((TPW,), jnp.int32),
        pltpu.VMEM((D,), jnp.float32),
        pltpu.VMEM((D,), jnp.float32),
        pltpu.SemaphoreType.DMA,
        pltpu.SemaphoreType.DMA,
        pltpu.SemaphoreType.DMA,
        pltpu.SemaphoreType.DMA,
    ],
    compiler_params=pltpu.CompilerParams(needs_layout_passes=False),
)(_ln_body)


def kernel(input_ids, token_type_ids, word_emb, pos_emb, type_emb, gamma, beta):
    b, l = input_ids.shape
    ids = input_ids.reshape(-1).astype(jnp.int32)
    tts = token_type_ids.reshape(-1).astype(jnp.int32)
    out = _ln_kernel(ids, tts, word_emb, pos_emb, type_emb, gamma, beta)
    return out.reshape(b, l, D)


# speedup vs baseline: 1.4060x; 1.4060x over previous
import functools

import jax
import jax.numpy as jnp
from jax import lax
from jax.experimental import pallas as pl
from jax.experimental.pallas import tpu as pltpu
from jax.experimental.pallas import tpu_sc as plsc

D = 128
SEQ = 200
TOK = 1024 * SEQ
NC, NS = 2, 16
NW = NC * NS
TPW = TOK // NW
C = 128
NCHUNK = TPW // C
LN_EPS = 1e-12
KVEC = D // 16


def _ln_body(ids, tts, word, pos, typ, gamma, beta, out,
             comb, typev, rows, idxall, ttall, gv, bv, sem):
    wid = lax.axis_index("s") * NC + lax.axis_index("c")
    base0 = wid * TPW

    pltpu.sync_copy(ids.at[pl.ds(base0, TPW)], idxall)
    pltpu.sync_copy(tts.at[pl.ds(base0, TPW)], ttall)
    pltpu.sync_copy(pos.at[pl.ds(0, SEQ)], comb.at[pl.ds(0, SEQ)])
    pltpu.sync_copy(pos.at[pl.ds(0, SEQ)], comb.at[pl.ds(SEQ, SEQ)])
    pltpu.sync_copy(typ, typev)
    pltpu.sync_copy(gamma, gv)
    pltpu.sync_copy(beta, bv)

    @plsc.parallel_loop(0, SEQ, 1, unroll=2)
    def add_type(l):
        for k in range(KVEC):
            s = pl.ds(k * 16, 16)
            comb[l, s] = comb[l, s] + typev[0, s]
            comb[SEQ + l, s] = comb[SEQ + l, s] + typev[1, s]

    lanes = lax.iota(jnp.int32, 16)
    perms = [lanes ^ sh for sh in (1, 2, 4, 8)]

    _gdn = lax.GatherDimensionNumbers(
        offset_dims=(), collapsed_slice_dims=(0,), start_index_map=(0,))

    def _perm(v, p):
        return lax.gather(v, p[:, None], dimension_numbers=_gdn,
                          slice_sizes=(1,),
                          mode=lax.GatherScatterMode.PROMISE_IN_BOUNDS)

    def xsum(v):
        for p in perms:
            v = v + _perm(v, p)
        return v

    gvs = [gv[pl.ds(k * 16, 16)] for k in range(KVEC)]
    bvs = [bv[pl.ds(k * 16, 16)] for k in range(KVEC)]

    def chunk_body(c, carry):
        coff = c * C
        pltpu.async_copy(
            word.at[idxall.at[pl.ds(coff, C)]], rows, sem).wait()
        pos0 = lax.rem(base0 + coff, SEQ)

        @plsc.parallel_loop(0, C, 1, unroll=4)
        def tok_body(t):
            tsplat = jnp.full((16,), coff + t, jnp.int32)
            ttk = plsc.load_gather(ttall, [tsplat])
            p = lax.rem(pos0 + t, SEQ)
            arow = ttk * SEQ + p
            xs = []
            for k in range(KVEC):
                ad = plsc.load_gather(comb, [arow, lanes + (k * 16)])
                xs.append(rows[t, pl.ds(k * 16, 16)] + ad)
            s = xs[0]
            for k in range(1, KVEC):
                s = s + xs[k]
            sq = xs[0] * xs[0]
            for k in range(1, KVEC):
                sq = sq + xs[k] * xs[k]
            meanv = xsum(s) * (1.0 / D)
            varv = xsum(sq) * (1.0 / D) - meanv * meanv + LN_EPS
            yi = jnp.int32(0x5F3759DF) - lax.shift_right_arithmetic(
                plsc.bitcast(varv, jnp.int32), 1)
            y = plsc.bitcast(yi, jnp.float32)
            for _ in range(2):
                y = y * (1.5 - 0.5 * varv * y * y)
            for k in range(KVEC):
                rows[t, pl.ds(k * 16, 16)] = (xs[k] - meanv) * (gvs[k] * y) + bvs[k]

        pltpu.sync_copy(rows, out.at[pl.ds(base0 + coff, C)])
        return carry

    lax.fori_loop(0, NCHUNK, chunk_body, 0)


_ln_kernel = functools.partial(
    pl.kernel,
    out_type=jax.ShapeDtypeStruct((TOK, D), jnp.float32),
    mesh=plsc.VectorSubcoreMesh(core_axis_name="c", subcore_axis_name="s"),
    scratch_types=[
        pltpu.VMEM((2 * SEQ, D), jnp.float32),
        pltpu.VMEM((2, D), jnp.float32),
        pltpu.VMEM((C, D), jnp.float32),
        pltpu.VMEM((TPW,), jnp.int32),
        pltpu.VMEM((TPW,), jnp.int32),
        pltpu.VMEM((D,), jnp.float32),
        pltpu.VMEM((D,), jnp.float32),
        pltpu.SemaphoreType.DMA,
    ],
    compiler_params=pltpu.CompilerParams(needs_layout_passes=False),
)(_ln_body)


def kernel(input_ids, token_type_ids, word_emb, pos_emb, type_emb, gamma, beta):
    b, l = input_ids.shape
    ids = input_ids.reshape(-1).astype(jnp.int32)
    tts = token_type_ids.reshape(-1).astype(jnp.int32)
    out = _ln_kernel(ids, tts, word_emb, pos_emb, type_emb, gamma, beta)
    return out.reshape(b, l, D)
